# baseline (speedup 1.0000x reference)
import functools

import jax
import jax.numpy as jnp
from jax import lax
from jax.experimental import pallas as pl
from jax.experimental.pallas import tpu as pltpu
from jax.experimental.pallas import tpu_sc as plsc

_NC = 2
_NS = 16
_NW = _NC * _NS
_W = 128
_NBUF = 4


def _transpose_pad(table_t, V, D, DP):
    X = 8192

    def body(in_ref, out_ref):
        out_ref[:, :D] = in_ref[...].T

    return pl.pallas_call(
        body,
        grid=(pl.cdiv(V, X),),
        in_specs=[pl.BlockSpec((D, X), lambda i: (0, i))],
        out_specs=pl.BlockSpec((X, DP), lambda i: (i, 0)),
        out_shape=jax.ShapeDtypeStruct((V, DP), jnp.float32),
        compiler_params=pltpu.CompilerParams(
            dimension_semantics=("parallel",)
        ),
    )(table_t)


def _gather(table_p, idx, n_chunk, DP):
    N_part = _NW * n_chunk * _W
    mesh = plsc.VectorSubcoreMesh(core_axis_name="c", subcore_axis_name="s")

    @functools.partial(
        pl.kernel,
        mesh=mesh,
        out_type=jax.ShapeDtypeStruct((N_part, DP), table_p.dtype),
        scratch_types=[
            pltpu.VMEM((n_chunk, _W), jnp.int32),
            pltpu.VMEM((_NBUF, _W, DP), jnp.float32),
            [pltpu.SemaphoreType.DMA] * _NBUF,
            [pltpu.SemaphoreType.DMA] * _NBUF,
        ],
    )
    def gather_kernel(table_hbm, idx_hbm, out_hbm, idx_v, rows_v, gsems, wsems):
        wid = lax.axis_index("s") * _NC + lax.axis_index("c")
        base = wid * (n_chunk * _W)
        pltpu.sync_copy(idx_hbm.at[wid], idx_v)

        def start_gather(j, b):
            pltpu.async_copy(
                table_hbm.at[idx_v.at[j]], rows_v.at[b], gsems[b]
            )

        def start_write(j, b):
            pltpu.async_copy(
                rows_v.at[b], out_hbm.at[pl.ds(base + j * _W, _W)], wsems[b]
            )

        start_gather(0, 0)
        start_gather(1, 1)

        @pl.loop(0, n_chunk, step=_NBUF)
        def _(j0):
            for b in range(_NBUF):
                j = j0 + b
                bg = (b + 2) % _NBUF

                def wait_write(bb=bg):
                    pltpu.make_async_copy(
                        rows_v.at[bb],
                        out_hbm.at[pl.ds(base, _W)],
                        wsems[bb],
                    ).wait()

                if b < 2:
                    @pl.when(j0 > 0)
                    def _():
                        wait_write()
                else:
                    wait_write()

                @pl.when(j + 2 < n_chunk)
                def _():
                    start_gather(j + 2, bg)

                pltpu.make_async_copy(
                    table_hbm.at[idx_v.at[j]], rows_v.at[b], gsems[b]
                ).wait()
                start_write(j, b)

        for b in (_NBUF - 2, _NBUF - 1):
            pltpu.make_async_copy(
                rows_v.at[b], out_hbm.at[pl.ds(base, _W)], wsems[b]
            ).wait()

    return gather_kernel(table_p, idx)


def kernel(x, table):
    B, S = x.shape
    V, D = table.shape
    DP = 128
    N = B * S
    HALVES = 2
    n_chunk = N // (HALVES * _NW * _W)
    assert N == HALVES * _NW * n_chunk * _W and n_chunk % _NBUF == 0

    table_p = _transpose_pad(table.T, V, D, DP)
    Sh = S // HALVES

    def finalize(g_h, h, prev):
        g_v = g_h.reshape(B, Sh * DP)
        out_sds = jax.ShapeDtypeStruct((S * D, B), jnp.float32)

        def body(in_ref, *rest):
            out_ref = rest[-1]
            out_ref[...] = in_ref[:, :D].T

        operands = [g_v]
        in_specs = [pl.BlockSpec((B, DP), lambda i: (0, i))]
        io_aliases = {}
        if prev is not None:
            operands.append(prev)
            in_specs.append(pl.BlockSpec(memory_space=pl.ANY))
            io_aliases = {1: 0}

        return pl.pallas_call(
            body,
            grid=(Sh,),
            in_specs=in_specs,
            out_specs=pl.BlockSpec((D, B), lambda i, h=h: (h * Sh + i, 0)),
            out_shape=out_sds,
            input_output_aliases=io_aliases,
            compiler_params=pltpu.CompilerParams(
                dimension_semantics=("arbitrary",)
            ),
        )(*operands)

    acc = None
    for h in range(HALVES):
        idx_h = x[:, h * Sh:(h + 1) * Sh].reshape(_NW, n_chunk, _W)
        g_h = _gather(table_p, idx_h, n_chunk, DP)
        acc = finalize(g_h, h, acc)

    return acc.reshape(S, D, B).transpose(2, 0, 1)


# speedup vs baseline: 1.5938x; 1.5938x over previous
import functools

import jax
import jax.numpy as jnp
from jax import lax
from jax.experimental import pallas as pl
from jax.experimental.pallas import tpu as pltpu
from jax.experimental.pallas import tpu_sc as plsc

_NC = 2
_NS = 16
_NW = _NC * _NS
_W = 128
_NBUF = 4


def _transpose_pad(table_t, V, D, DP):
    X = 8192

    def body(in_ref, out_ref):
        out_ref[:, :D] = in_ref[...].T

    return pl.pallas_call(
        body,
        grid=(pl.cdiv(V, X),),
        in_specs=[pl.BlockSpec((D, X), lambda i: (0, i))],
        out_specs=pl.BlockSpec((X, DP), lambda i: (i, 0)),
        out_shape=jax.ShapeDtypeStruct((V, DP), jnp.float32),
        compiler_params=pltpu.CompilerParams(
            dimension_semantics=("parallel",)
        ),
    )(table_t)


def kernel(x, table):
    B, S = x.shape
    V, D = table.shape
    DP = 128
    N = B * S
    n_chunk = N // (_NW * _W)
    assert N == _NW * n_chunk * _W and n_chunk % _NBUF == 0

    table_p = _transpose_pad(table.T, V, D, DP)
    idx = x.reshape(_NW, n_chunk, _W)
    mesh = plsc.VectorSubcoreMesh(core_axis_name="c", subcore_axis_name="s")

    @functools.partial(
        pl.kernel,
        mesh=mesh,
        out_type=jax.ShapeDtypeStruct((N, DP), table.dtype),
        scratch_types=[
            pltpu.VMEM((n_chunk, _W), jnp.int32),
            pltpu.VMEM((_NBUF, _W, DP), jnp.float32),
            [pltpu.SemaphoreType.DMA] * _NBUF,
            [pltpu.SemaphoreType.DMA] * _NBUF,
        ],
    )
    def gather_kernel(table_hbm, idx_hbm, out_hbm, idx_v, rows_v, gsems, wsems):
        wid = lax.axis_index("s") * _NC + lax.axis_index("c")
        base = wid * (n_chunk * _W)
        pltpu.sync_copy(idx_hbm.at[wid], idx_v)

        def start_gather(j, b):
            pltpu.async_copy(
                table_hbm.at[idx_v.at[j]], rows_v.at[b], gsems[b]
            )

        def start_write(j, b):
            pltpu.async_copy(
                rows_v.at[b], out_hbm.at[pl.ds(base + j * _W, _W)], wsems[b]
            )

        start_gather(0, 0)
        start_gather(1, 1)

        @pl.loop(0, n_chunk, step=_NBUF)
        def _(j0):
            for b in range(_NBUF):
                j = j0 + b
                bg = (b + 2) % _NBUF

                def wait_write(bb=bg):
                    pltpu.make_async_copy(
                        rows_v.at[bb],
                        out_hbm.at[pl.ds(base, _W)],
                        wsems[bb],
                    ).wait()

                if b < 2:
                    @pl.when(j0 > 0)
                    def _():
                        wait_write()
                else:
                    wait_write()

                @pl.when(j + 2 < n_chunk)
                def _():
                    start_gather(j + 2, bg)

                pltpu.make_async_copy(
                    table_hbm.at[idx_v.at[j]], rows_v.at[b], gsems[b]
                ).wait()
                start_write(j, b)

        for b in (_NBUF - 2, _NBUF - 1):
            pltpu.make_async_copy(
                rows_v.at[b], out_hbm.at[pl.ds(base, _W)], wsems[b]
            ).wait()

    out = gather_kernel(table_p, idx)
    return out[:, :D].reshape(B, S, D)


# speedup vs baseline: 1.6386x; 1.0281x over previous
import functools

import jax
import jax.numpy as jnp
from jax import lax
from jax.experimental import pallas as pl
from jax.experimental.pallas import tpu as pltpu
from jax.experimental.pallas import tpu_sc as plsc

_NC = 2
_NS = 16
_NW = _NC * _NS
_W = 128
_NBUF = 4


def _transpose_pad(table_t, V, D, DP):
    X = 16384

    def body(in_ref, out_ref):
        out_ref[:, :D] = in_ref[...].T

    return pl.pallas_call(
        body,
        grid=(pl.cdiv(V, X),),
        in_specs=[pl.BlockSpec((D, X), lambda i: (0, i))],
        out_specs=pl.BlockSpec((X, DP), lambda i: (i, 0)),
        out_shape=jax.ShapeDtypeStruct((V, DP), jnp.float32),
        compiler_params=pltpu.CompilerParams(
            dimension_semantics=("parallel",)
        ),
    )(table_t)


def kernel(x, table):
    B, S = x.shape
    V, D = table.shape
    DP = 128
    N = B * S
    n_chunk = N // (_NW * _W)
    assert N == _NW * n_chunk * _W and n_chunk % _NBUF == 0

    table_p = _transpose_pad(table.T, V, D, DP)
    idx = x.reshape(_NW, n_chunk, _W)
    mesh = plsc.VectorSubcoreMesh(core_axis_name="c", subcore_axis_name="s")

    @functools.partial(
        pl.kernel,
        mesh=mesh,
        out_type=jax.ShapeDtypeStruct((N, DP), table.dtype),
        scratch_types=[
            pltpu.VMEM((n_chunk, _W), jnp.int32),
            pltpu.VMEM((_NBUF, _W, DP), jnp.float32),
            [pltpu.SemaphoreType.DMA] * _NBUF,
            [pltpu.SemaphoreType.DMA] * _NBUF,
        ],
    )
    def gather_kernel(table_hbm, idx_hbm, out_hbm, idx_v, rows_v, gsems, wsems):
        wid = lax.axis_index("s") * _NC + lax.axis_index("c")
        base = wid * (n_chunk * _W)
        pltpu.sync_copy(idx_hbm.at[wid], idx_v)

        def start_gather(j, b):
            pltpu.async_copy(
                table_hbm.at[idx_v.at[j]], rows_v.at[b], gsems[b]
            )

        def start_write(j, b):
            pltpu.async_copy(
                rows_v.at[b], out_hbm.at[pl.ds(base + j * _W, _W)], wsems[b]
            )

        start_gather(0, 0)
        start_gather(1, 1)

        @pl.loop(0, n_chunk, step=_NBUF)
        def _(j0):
            for b in range(_NBUF):
                j = j0 + b
                bg = (b + 2) % _NBUF

                def wait_write(bb=bg):
                    pltpu.make_async_copy(
                        rows_v.at[bb],
                        out_hbm.at[pl.ds(base, _W)],
                        wsems[bb],
                    ).wait()

                if b < 2:
                    @pl.when(j0 > 0)
                    def _():
                        wait_write()
                else:
                    wait_write()

                @pl.when(j + 2 < n_chunk)
                def _():
                    start_gather(j + 2, bg)

                pltpu.make_async_copy(
                    table_hbm.at[idx_v.at[j]], rows_v.at[b], gsems[b]
                ).wait()
                start_write(j, b)

        for b in (_NBUF - 2, _NBUF - 1):
            pltpu.make_async_copy(
                rows_v.at[b], out_hbm.at[pl.ds(base, _W)], wsems[b]
            ).wait()

    out = gather_kernel(table_p, idx)
    return out[:, :D].reshape(B, S, D)


# speedup vs baseline: 1.6542x; 1.0095x over previous
import functools

import jax
import jax.numpy as jnp
from jax import lax
from jax.experimental import pallas as pl
from jax.experimental.pallas import tpu as pltpu
from jax.experimental.pallas import tpu_sc as plsc

_NC = 2
_NS = 16
_NW = _NC * _NS
_W = 128
_NBUF = 4


def _transpose_pad(table_t, V, D, DP):
    X = 32768

    def body(in_ref, out_ref):
        out_ref[:, :D] = in_ref[...].T

    return pl.pallas_call(
        body,
        grid=(pl.cdiv(V, X),),
        in_specs=[pl.BlockSpec((D, X), lambda i: (0, i))],
        out_specs=pl.BlockSpec((X, DP), lambda i: (i, 0)),
        out_shape=jax.ShapeDtypeStruct((V, DP), jnp.float32),
        compiler_params=pltpu.CompilerParams(
            dimension_semantics=("parallel",)
        ),
    )(table_t)


def kernel(x, table):
    B, S = x.shape
    V, D = table.shape
    DP = 128
    N = B * S
    n_chunk = N // (_NW * _W)
    assert N == _NW * n_chunk * _W and n_chunk % _NBUF == 0

    table_p = _transpose_pad(table.T, V, D, DP)
    idx = x.reshape(_NW, n_chunk, _W)
    mesh = plsc.VectorSubcoreMesh(core_axis_name="c", subcore_axis_name="s")

    @functools.partial(
        pl.kernel,
        mesh=mesh,
        out_type=jax.ShapeDtypeStruct((N, DP), table.dtype),
        scratch_types=[
            pltpu.VMEM((n_chunk, _W), jnp.int32),
            pltpu.VMEM((_NBUF, _W, DP), jnp.float32),
            [pltpu.SemaphoreType.DMA] * _NBUF,
            [pltpu.SemaphoreType.DMA] * _NBUF,
        ],
    )
    def gather_kernel(table_hbm, idx_hbm, out_hbm, idx_v, rows_v, gsems, wsems):
        wid = lax.axis_index("s") * _NC + lax.axis_index("c")
        base = wid * (n_chunk * _W)
        pltpu.sync_copy(idx_hbm.at[wid], idx_v)

        def start_gather(j, b):
            pltpu.async_copy(
                table_hbm.at[idx_v.at[j]], rows_v.at[b], gsems[b]
            )

        def start_write(j, b):
            pltpu.async_copy(
                rows_v.at[b], out_hbm.at[pl.ds(base + j * _W, _W)], wsems[b]
            )

        start_gather(0, 0)
        start_gather(1, 1)

        @pl.loop(0, n_chunk, step=_NBUF)
        def _(j0):
            for b in range(_NBUF):
                j = j0 + b
                bg = (b + 2) % _NBUF

                def wait_write(bb=bg):
                    pltpu.make_async_copy(
                        rows_v.at[bb],
                        out_hbm.at[pl.ds(base, _W)],
                        wsems[bb],
                    ).wait()

                if b < 2:
                    @pl.when(j0 > 0)
                    def _():
                        wait_write()
                else:
                    wait_write()

                @pl.when(j + 2 < n_chunk)
                def _():
                    start_gather(j + 2, bg)

                pltpu.make_async_copy(
                    table_hbm.at[idx_v.at[j]], rows_v.at[b], gsems[b]
                ).wait()
                start_write(j, b)

        for b in (_NBUF - 2, _NBUF - 1):
            pltpu.make_async_copy(
                rows_v.at[b], out_hbm.at[pl.ds(base, _W)], wsems[b]
            ).wait()

    out = gather_kernel(table_p, idx)
    return out[:, :D].reshape(B, S, D)


# speedup vs baseline: 1.6542x; 1.0001x over previous
import functools

import jax
import jax.numpy as jnp
from jax import lax
from jax.experimental import pallas as pl
from jax.experimental.pallas import tpu as pltpu
from jax.experimental.pallas import tpu_sc as plsc

_NC = 2
_NS = 16
_NW = _NC * _NS
_W = 128
_NBUF = 4


def _transpose_pad(table_t, V, D, DP):
    X = 32768

    def body(in_ref, out_ref):
        out_ref[:, :D] = in_ref[...].T

    return pl.pallas_call(
        body,
        grid=(pl.cdiv(V, X),),
        in_specs=[pl.BlockSpec((D, X), lambda i: (0, i))],
        out_specs=pl.BlockSpec((X, DP), lambda i: (i, 0)),
        out_shape=jax.ShapeDtypeStruct((V, DP), jnp.float32),
        compiler_params=pltpu.CompilerParams(
            dimension_semantics=("parallel",)
        ),
    )(table_t)


def kernel(x, table):
    B, S = x.shape
    V, D = table.shape
    DP = 128
    N = B * S
    n_chunk = N // (_NW * _W)
    assert N == _NW * n_chunk * _W and n_chunk % _NBUF == 0

    table_p = _transpose_pad(table.T, V, D, DP)
    idx = x.reshape(_NW, n_chunk, _W)
    mesh = plsc.VectorSubcoreMesh(core_axis_name="c", subcore_axis_name="s")

    @functools.partial(
        pl.kernel,
        mesh=mesh,
        out_type=jax.ShapeDtypeStruct((N, DP), table.dtype),
        scratch_types=[
            pltpu.VMEM((n_chunk, _W), jnp.int32),
            pltpu.VMEM((_NBUF, _W, DP), jnp.float32),
            [pltpu.SemaphoreType.DMA] * _NBUF,
            [pltpu.SemaphoreType.DMA] * _NBUF,
        ],
    )
    def gather_kernel(table_hbm, idx_hbm, out_hbm, idx_v, rows_v, gsems, wsems):
        wid = lax.axis_index("s") * _NC + lax.axis_index("c")
        base = wid * (n_chunk * _W)
        pltpu.sync_copy(idx_hbm.at[wid], idx_v)

        def start_gather(j, b):
            pltpu.async_copy(
                table_hbm.at[idx_v.at[j]], rows_v.at[b], gsems[b]
            )

        def start_write(j, b):
            pltpu.async_copy(
                rows_v.at[b], out_hbm.at[pl.ds(base + j * _W, _W)], wsems[b]
            )

        start_gather(0, 0)
        start_gather(1, 1)
        start_gather(2, 2)

        @pl.loop(0, n_chunk, step=_NBUF)
        def _(j0):
            for b in range(_NBUF):
                j = j0 + b
                bw = (b + 3) % _NBUF

                def wait_write(bb=bw):
                    pltpu.make_async_copy(
                        rows_v.at[bb],
                        out_hbm.at[pl.ds(base, _W)],
                        wsems[bb],
                    ).wait()

                if b == 0:
                    @pl.when(j0 > 0)
                    def _():
                        wait_write()
                else:
                    wait_write()

                @pl.when(j + 3 < n_chunk)
                def _():
                    start_gather(j + 3, bw)

                pltpu.make_async_copy(
                    table_hbm.at[idx_v.at[j]], rows_v.at[b], gsems[b]
                ).wait()
                start_write(j, b)

        pltpu.make_async_copy(
            rows_v.at[_NBUF - 1], out_hbm.at[pl.ds(base, _W)], wsems[_NBUF - 1]
        ).wait()

    out = gather_kernel(table_p, idx)
    return out[:, :D].reshape(B, S, D)
